# Initial kernel scaffold; baseline (speedup 1.0000x reference)
import jax
import jax.numpy as jnp
from jax.experimental import pallas as pl


def kernel(x, params, neigh1, neigh2, neigh3, neigh4, pool1, pool2):
    raise NotImplementedError("write your pallas kernel here")


# speedup vs baseline: 1.9394x; 1.9394x over previous
import functools

import jax
import jax.numpy as jnp
from jax import lax
from jax.experimental import pallas as pl
from jax.experimental.pallas import tpu as pltpu
from jax.experimental.pallas import tpu_sc as plsc

NC = 2
NS = 16
NW = NC * NS
LANES = 16


def _leaky(v):
    return jnp.where(v >= 0, v, 0.1 * v)


def _round_up(n, m):
    return (n + m - 1) // m * m


@functools.partial(jax.jit, static_argnames=("nchunks", "c", "k", "d"))
def _sc_gather_sum(table, idx_flat, nchunks, c, k, d):
    np_rows = nchunks * c * NW
    p = nchunks * c
    mesh = plsc.VectorSubcoreMesh(core_axis_name="c", subcore_axis_name="s")

    @functools.partial(
        pl.kernel,
        out_type=jax.ShapeDtypeStruct((np_rows, d), jnp.float32),
        mesh=mesh,
        compiler_params=pltpu.CompilerParams(use_tc_tiling_on_sc=False),
        scratch_types=[
            pltpu.VMEM((c * k,), jnp.int32),
            pltpu.VMEM((c * k, d), jnp.float32),
            pltpu.VMEM((c, d), jnp.float32),
            pltpu.SemaphoreType.DMA,
        ],
    )
    def kern(table_hbm, idx_hbm, out_hbm, idx_v, rows_v, out_v, sem):
        wid = lax.axis_index("s") * NC + lax.axis_index("c")
        base = wid * p

        def chunk(i, carry):
            off = base + i * c
            pltpu.sync_copy(idx_hbm.at[pl.ds(off * k, c * k)], idx_v)
            pltpu.async_copy(table_hbm.at[idx_v], rows_v, sem).wait()

            def point(cc, carry2):
                for g in range(d // LANES):
                    sl = pl.ds(g * LANES, LANES)
                    vals = [rows_v[cc * k + kk, sl] for kk in range(k)]
                    m = k
                    while m > 1:
                        m //= 2
                        vals = [vals[j] + vals[j + m] for j in range(m)]
                    out_v[cc, sl] = vals[0]
                return carry2

            lax.fori_loop(0, c, point, 0)
            pltpu.sync_copy(out_v, out_hbm.at[pl.ds(off, c)])
            return carry

        lax.fori_loop(0, nchunks, chunk, 0)

    return kern(table, idx_flat)


def _gather_sum(table, idx):
    n, k = idx.shape
    d = table.shape[1]
    c = 128 // k
    np_rows = _round_up(n, NW * c)
    nchunks = np_rows // (NW * c)
    idx_p = jnp.pad(idx, ((0, np_rows - n), (0, 0))).reshape(-1)
    out = _sc_gather_sum(table, idx_p, nchunks, c, k, d)
    return out[:n]


@functools.partial(jax.jit, static_argnames=("c", "d"))
def _sc_gather_rows(table, idx_p, c, d):
    np_rows = c * NW
    mesh = plsc.VectorSubcoreMesh(core_axis_name="c", subcore_axis_name="s")

    @functools.partial(
        pl.kernel,
        out_type=jax.ShapeDtypeStruct((np_rows, d), jnp.float32),
        mesh=mesh,
        compiler_params=pltpu.CompilerParams(use_tc_tiling_on_sc=False),
        scratch_types=[
            pltpu.VMEM((c,), jnp.int32),
            pltpu.VMEM((c, d), jnp.float32),
            pltpu.SemaphoreType.DMA,
        ],
    )
    def kern(table_hbm, idx_hbm, out_hbm, idx_v, rows_v, sem):
        wid = lax.axis_index("s") * NC + lax.axis_index("c")
        base = wid * c
        pltpu.sync_copy(idx_hbm.at[pl.ds(base, c)], idx_v)
        pltpu.async_copy(table_hbm.at[idx_v], rows_v, sem).wait()
        pltpu.sync_copy(rows_v, out_hbm.at[pl.ds(base, c)])

    return kern(table, idx_p)


def _gather_rows(table, idx):
    n = idx.shape[0]
    d = table.shape[1]
    c = _round_up(-(-n // NW), 8)
    assert c <= 128
    np_rows = c * NW
    idx_p = jnp.pad(idx, (0, np_rows - n))
    out = _sc_gather_rows(table, idx_p, c, d)
    return out[:n]


def _dot(a, b):
    return jnp.dot(a, b, precision=lax.Precision.DEFAULT,
                   preferred_element_type=jnp.float32)


def _stage1_body(s_ref, w1h_ref, w1t_ref, h_ref, t_ref):
    h = _leaky(s_ref[...][:, :1] * w1h_ref[...])
    h_ref[...] = h
    t_ref[...] = _leaky(_dot(h, w1t_ref[...]))


def _tc_stage1(s1, w1h, w1t, br):
    n = s1.shape[0]
    grid = (n // br,)
    return pl.pallas_call(
        _stage1_body,
        grid=grid,
        in_specs=[
            pl.BlockSpec((br, s1.shape[1]), lambda r: (r, 0)),
            pl.BlockSpec(w1h.shape, lambda r: (0, 0)),
            pl.BlockSpec(w1t.shape, lambda r: (0, 0)),
        ],
        out_specs=[
            pl.BlockSpec((br, w1h.shape[1]), lambda r: (r, 0)),
            pl.BlockSpec((br, w1t.shape[1]), lambda r: (r, 0)),
        ],
        out_shape=[
            jax.ShapeDtypeStruct((n, w1h.shape[1]), jnp.float32),
            jax.ShapeDtypeStruct((n, w1t.shape[1]), jnp.float32),
        ],
    )(s1, w1h, w1t)


def _resblock_body(has_ws, has_w1n, *refs):
    i = 0
    x_ref = refs[i]; i += 1
    s_ref = refs[i]; i += 1
    wk_ref = refs[i]; i += 1
    w2_ref = refs[i]; i += 1
    ws_ref = None
    if has_ws:
        ws_ref = refs[i]; i += 1
    w1n_ref = None
    if has_w1n:
        w1n_ref = refs[i]; i += 1
    out_ref = refs[i]; i += 1

    x = x_ref[...]
    h = _leaky(_dot(s_ref[...], wk_ref[...]))
    h = _dot(h, w2_ref[...])
    sc = _dot(x, ws_ref[...]) if has_ws else x
    out = _leaky(h + sc)
    out_ref[...] = out
    if has_w1n:
        refs[i][...] = _leaky(_dot(out, w1n_ref[...]))


def _tc_resblock(x, s, wk, w2, ws=None, w1n=None, br=None):
    n, din = x.shape
    o = w2.shape[1]
    if br is None:
        br = n
    grid = (n // br,)
    ins = [x, s, wk, w2]
    in_specs = [
        pl.BlockSpec((br, din), lambda r: (r, 0)),
        pl.BlockSpec((br, wk.shape[0]), lambda r: (r, 0)),
        pl.BlockSpec(wk.shape, lambda r: (0, 0)),
        pl.BlockSpec(w2.shape, lambda r: (0, 0)),
    ]
    if ws is not None:
        ins.append(ws)
        in_specs.append(pl.BlockSpec(ws.shape, lambda r: (0, 0)))
    if w1n is not None:
        ins.append(w1n)
        in_specs.append(pl.BlockSpec(w1n.shape, lambda r: (0, 0)))
    out_specs = [pl.BlockSpec((br, o), lambda r: (r, 0))]
    out_shape = [jax.ShapeDtypeStruct((n, o), jnp.float32)]
    if w1n is not None:
        out_specs.append(pl.BlockSpec((br, w1n.shape[1]), lambda r: (r, 0)))
        out_shape.append(jax.ShapeDtypeStruct((n, w1n.shape[1]), jnp.float32))
    res = pl.pallas_call(
        functools.partial(_resblock_body, ws is not None, w1n is not None),
        grid=grid,
        in_specs=in_specs,
        out_specs=out_specs,
        out_shape=out_shape,
    )(*ins)
    if w1n is None:
        return res[0]
    return res


def _final_body(x_ref, s_ref, wk_ref, w2_ref, ws_ref, cw_ref, cg_ref, cb_ref,
                f_ref):
    h = _leaky(_dot(s_ref[...], wk_ref[...]))
    h = _dot(h, w2_ref[...])
    out = _leaky(h + _dot(x_ref[...], ws_ref[...]))
    z = _dot(out, cw_ref[...])
    mu = jnp.mean(z, axis=0, keepdims=True)
    var = jnp.mean((z - mu) ** 2, axis=0, keepdims=True)
    zn = (z - mu) / jnp.sqrt(var + 1e-5)
    f_ref[...] = _leaky(zn * cg_ref[...] + cb_ref[...])


def _tc_final(x, s, wk, w2, ws, cw, cg, cb):
    n = x.shape[0]
    return pl.pallas_call(
        _final_body,
        out_shape=jax.ShapeDtypeStruct((n, cw.shape[1]), jnp.float32),
    )(x, s, wk, w2, ws, cw, cg.reshape(1, -1), cb.reshape(1, -1))


def kernel(x, params, neigh1, neigh2, neigh3, neigh4, pool1, pool2):
    p = params
    kk = neigh1.shape[1]
    inv_k = 1.0 / kk

    x16 = jnp.broadcast_to(x, (x.shape[0], LANES))
    s1 = _gather_sum(x16, neigh1)[:, :1]
    h1, t1 = _tc_stage1(s1, p['kp1'] * inv_k, p['rb1']['W1'], br=2000)

    s2 = _gather_sum(t1, neigh1)
    h2, t2 = _tc_resblock(h1, s2, p['rb1']['Wk'] * inv_k, p['rb1']['W2'],
                          ws=p['rb1']['Ws'], w1n=p['ra1']['W1'], br=2000)
    s3 = _gather_sum(t2, neigh1)
    h3, tf1 = _tc_resblock(h2, s3, p['ra1']['Wk'] * inv_k, p['ra1']['W2'],
                           ws=None, w1n=p['rb2a']['W1'], br=2000)
    skip1 = h3

    cat1 = jnp.concatenate([h3, tf1], axis=1)
    g1 = _gather_rows(cat1, pool1)
    x2, t3 = g1[:, :h3.shape[1]], g1[:, h3.shape[1]:]

    s4 = _gather_sum(t3, neigh2)
    h4, t4 = _tc_resblock(x2, s4, p['rb2a']['Wk'] * inv_k, p['rb2a']['W2'],
                          ws=p['rb2a']['Ws'], w1n=p['rb2b']['W1'])
    s5 = _gather_sum(t4, neigh2)
    h5, t5 = _tc_resblock(h4, s5, p['rb2b']['Wk'] * inv_k, p['rb2b']['W2'],
                          ws=p['rb2b']['Ws'], w1n=p['ra2']['W1'])
    s6 = _gather_sum(t5, neigh2)
    h6, tf2 = _tc_resblock(h5, s6, p['ra2']['Wk'] * inv_k, p['ra2']['W2'],
                           ws=None, w1n=p['rb3a']['W1'])
    skip2 = h6

    cat2 = jnp.concatenate([h6, tf2], axis=1)
    g2 = _gather_rows(cat2, pool2)
    x3, t6 = g2[:, :h6.shape[1]], g2[:, h6.shape[1]:]

    s7 = _gather_sum(t6, neigh3)
    h7, t7 = _tc_resblock(x3, s7, p['rb3a']['Wk'] * inv_k, p['rb3a']['W2'],
                          ws=p['rb3a']['Ws'], w1n=p['rb3b']['W1'])
    s8 = _gather_sum(t7, neigh3)
    h8, t8 = _tc_resblock(h7, s8, p['rb3b']['Wk'] * inv_k, p['rb3b']['W2'],
                          ws=p['rb3b']['Ws'], w1n=p['ra3']['W1'])
    s9 = _gather_sum(t8, neigh3)
    h9, t9 = _tc_resblock(h8, s9, p['ra3']['Wk'] * inv_k, p['ra3']['W2'],
                          ws=None, w1n=p['rb4a']['W1'])
    skip3 = h9

    s10 = _gather_sum(t9, neigh4)
    h10, t10 = _tc_resblock(h9, s10, p['rb4a']['Wk'] * inv_k, p['rb4a']['W2'],
                            ws=p['rb4a']['Ws'], w1n=p['rb4b']['W1'])
    s11 = _gather_sum(t10, neigh4)
    final = _tc_final(h10, s11, p['rb4b']['Wk'] * inv_k, p['rb4b']['W2'],
                      p['rb4b']['Ws'], p['c4_W'], p['c4_g'], p['c4_b'])

    return final, skip1, skip2, skip3
